# baseline (speedup 1.0000x reference)
import functools

import jax
import jax.numpy as jnp
from jax import lax
from jax.experimental import pallas as pl
from jax.experimental.pallas import tpu as pltpu
from jax.experimental.pallas import tpu_sc as plsc

_NC = 2
_NS = 16
_NW = _NC * _NS
_L = 16
_EPS = 1e-5

_TOK = 20480
_FC = 26
_FN = 13
_D = 32
_ROW = (_FC + _FN) * _D
_SG_TOK = 64
_SG_CAT = _SG_TOK * _FC
_CHUNK = 128
_NCHUNK = _SG_CAT // _CHUNK
_TOK_PER_W = _TOK // _NW
_NSG = _TOK_PER_W // _SG_TOK
_STG = _SG_TOK * _ROW // 128


def _rsqrt_newton(z):
    i = jnp.int32(0x5F3759DF) - lax.shift_right_logical(
        lax.bitcast_convert_type(z, jnp.int32), jnp.int32(1))
    y = lax.bitcast_convert_type(i, jnp.float32)
    half = jnp.float32(0.5)
    threehalf = jnp.float32(1.5)
    for _ in range(3):
        y = y * (threehalf - half * z * y * y)
    return y


def _sc_all(idx2, tab128, xn2, nw, nb, lnp):
    mesh = plsc.VectorSubcoreMesh(core_axis_name="c", subcore_axis_name="s")

    @functools.partial(
        pl.kernel,
        out_type=jax.ShapeDtypeStruct((_TOK * _ROW // 128, 128), jnp.float32),
        mesh=mesh,
        compiler_params=pltpu.CompilerParams(
            use_tc_tiling_on_sc=False, needs_layout_passes=False),
        scratch_types=[
            pltpu.VMEM((_NCHUNK, 128), jnp.int32),
            pltpu.VMEM((_NCHUNK, 128), jnp.int32),
            pltpu.VMEM((_CHUNK, 128), jnp.float32),
            pltpu.VMEM((_STG, 128), jnp.float32),
            pltpu.VMEM((_SG_TOK, _FN), jnp.float32),
            pltpu.VMEM((_FN, _D), jnp.float32),
            pltpu.VMEM((_FN, _D), jnp.float32),
            pltpu.VMEM((4, _D), jnp.float32),
            pltpu.SemaphoreType.DMA,
        ],
    )
    def k(idx_hbm, tab_hbm, xn_hbm, nw_hbm, nb_hbm, lnp_hbm, out_hbm,
          idxc_v, idx4_v, packed_v, stage_v, xn_v, nw_v, nb_v, lnp_v, sem):
        wid = lax.axis_index("s") * _NC + lax.axis_index("c")
        iota = lax.iota(jnp.int32, _L)
        inv_d = jnp.float32(1.0 / _D)
        eps = jnp.float32(_EPS)

        pltpu.sync_copy(nw_hbm, nw_v)
        pltpu.sync_copy(nb_hbm, nb_v)
        pltpu.sync_copy(lnp_hbm, lnp_v)

        def splat(x):
            return jnp.full((_L,), x, jnp.int32)

        def supergroup(s, carry):
            irow0 = wid * (_NSG * _NCHUNK) + s * _NCHUNK
            pltpu.sync_copy(idx_hbm.at[pl.ds(irow0, _NCHUNK)], idxc_v)
            tok0 = wid * _TOK_PER_W + s * _SG_TOK
            pltpu.sync_copy(xn_hbm.at[pl.ds(tok0, _SG_TOK)], xn_v)

            def quarter(jj, c2):
                row = splat(jj // jnp.int32(8))
                col = splat((jj % jnp.int32(8)) * jnp.int32(_L)) + iota
                v = plsc.load_gather(idxc_v, [row, col])
                plsc.store_scatter(idx4_v, [row, col],
                                   lax.shift_right_logical(v, jnp.int32(2)))
                return c2

            lax.fori_loop(0, _NCHUNK * 8, quarter, 0)

            def cat_chunk(c, c2):
                pltpu.async_copy(tab_hbm.at[idx4_v.at[c]], packed_v,
                                 sem).wait()

                def cat_block(b, c3):
                    r0 = splat(c * _CHUNK + b * _L) + iota
                    raw = plsc.load_gather(
                        idxc_v, [splat(c), splat(b * _L) + iota])
                    col0 = (raw & jnp.int32(3)) * jnp.int32(_D)
                    rows = splat(b * _L) + iota
                    t_vec = r0 // jnp.int32(_FC)
                    f_vec = r0 % jnp.int32(_FC)
                    lin0 = t_vec * jnp.int32(_ROW) + f_vec * jnp.int32(_D)

                    def p1(d8, acc):
                        sm, sq = acc
                        for dd in range(4):
                            d = d8 * 4 + dd
                            v = plsc.load_gather(
                                packed_v, [rows, col0 + splat(d)])
                            sm = sm + v
                            sq = sq + v * v
                        return sm, sq

                    zero = jnp.zeros((_L,), jnp.float32)
                    sm, sq = lax.fori_loop(0, 8, p1, (zero, zero))
                    mean = sm * inv_d
                    var = sq * inv_d - mean * mean
                    rstd = _rsqrt_newton(var + eps)

                    def p2(d8, c4):
                        for dd in range(4):
                            d = d8 * 4 + dd
                            dsp = splat(d)
                            v = plsc.load_gather(
                                packed_v, [rows, col0 + dsp])
                            g = plsc.load_gather(lnp_v, [splat(0), dsp])
                            bta = plsc.load_gather(lnp_v, [splat(1), dsp])
                            y = (v - mean) * rstd * g + bta
                            l = lin0 + dsp
                            plsc.store_scatter(
                                stage_v,
                                [lax.shift_right_logical(l, jnp.int32(7)),
                                 l & jnp.int32(127)], y)
                        return c4

                    lax.fori_loop(0, 8, p2, 0)
                    return c3

                lax.fori_loop(0, _CHUNK // _L, cat_block, 0)
                return c2

            lax.fori_loop(0, _NCHUNK, cat_chunk, 0)

            def num_block(kk, c2):
                tb = kk // jnp.int32(_FN)
                f = kk % jnp.int32(_FN)
                t_loc = splat(tb * _L) + iota
                x_vec = plsc.load_gather(xn_v, [t_loc, splat(f)])
                lin0 = t_loc * jnp.int32(_ROW) + splat(
                    (jnp.int32(_FC) + f) * jnp.int32(_D))

                def p1(d8, acc):
                    sm, sq = acc
                    for dd in range(4):
                        d = d8 * 4 + dd
                        wv = plsc.load_gather(nw_v, [splat(f), splat(d)])
                        bv = plsc.load_gather(nb_v, [splat(f), splat(d)])
                        v = x_vec * wv + bv
                        sm = sm + v
                        sq = sq + v * v
                    return sm, sq

                zero = jnp.zeros((_L,), jnp.float32)
                sm, sq = lax.fori_loop(0, 8, p1, (zero, zero))
                mean = sm * inv_d
                var = sq * inv_d - mean * mean
                rstd = _rsqrt_newton(var + eps)

                def p2(d8, c4):
                    for dd in range(4):
                        d = d8 * 4 + dd
                        dsp = splat(d)
                        wv = plsc.load_gather(nw_v, [splat(f), dsp])
                        bv = plsc.load_gather(nb_v, [splat(f), dsp])
                        v = x_vec * wv + bv
                        g = plsc.load_gather(lnp_v, [splat(2), dsp])
                        bta = plsc.load_gather(lnp_v, [splat(3), dsp])
                        y = (v - mean) * rstd * g + bta
                        l = lin0 + dsp
                        plsc.store_scatter(
                            stage_v,
                            [lax.shift_right_logical(l, jnp.int32(7)),
                             l & jnp.int32(127)], y)
                    return c4

                lax.fori_loop(0, 8, p2, 0)
                return c2

            lax.fori_loop(0, (_SG_TOK // _L) * _FN, num_block, 0)

            orow0 = wid * (_NSG * _STG) + s * _STG
            pltpu.sync_copy(stage_v, out_hbm.at[pl.ds(orow0, _STG)])
            return carry

        lax.fori_loop(0, _NSG, supergroup, 0)

    return k(idx2, tab128, xn2, nw, nb, lnp)


def kernel(x_cat, x_num, tables, num_w, num_b, cat_ln_g, cat_ln_b, num_ln_g,
           num_ln_b):
    B, O, Fc = x_cat.shape
    Fn = x_num.shape[-1]
    V, D = tables.shape[1], tables.shape[2]

    idx = x_cat.reshape(B * O, Fc).astype(jnp.int32) + jnp.arange(
        Fc, dtype=jnp.int32) * jnp.int32(V)
    lnp = jnp.stack([cat_ln_g, cat_ln_b, num_ln_g, num_ln_b])
    out = _sc_all(
        idx.reshape(B * O * Fc // 128, 128),
        tables.reshape(Fc * V * D // 128, 128),
        x_num.reshape(B * O, Fn),
        num_w, num_b, lnp,
    )
    return out.reshape(B, O, Fc + Fn, D)


# speedup vs baseline: 1.5031x; 1.5031x over previous
import functools

import jax
import jax.numpy as jnp
from jax import lax
from jax.experimental import pallas as pl
from jax.experimental.pallas import tpu as pltpu
from jax.experimental.pallas import tpu_sc as plsc

_NC = 2
_NS = 16
_NW = _NC * _NS
_L = 16
_EPS = 1e-5

_B = 1024
_O = 20
_FC = 26
_FN = 13
_D = 32
_BC = 128
_NBC = _B // _BC
_NBLK = _O * _NBC
_BLK_PER_W = _NBLK // _NW
_NSUB = _BC // _L

_OFF_NW = 0
_OFF_NB = _FN * _D
_OFF_GC = 2 * _FN * _D
_OFF_BCZ = _OFF_GC + _D
_OFF_GN = _OFF_GC + 2 * _D
_OFF_BN = _OFF_GC + 3 * _D


def _rsqrt_newton(z):
    i = jnp.int32(0x5F3759DF) - lax.shift_right_logical(
        lax.bitcast_convert_type(z, jnp.int32), jnp.int32(1))
    y = lax.bitcast_convert_type(i, jnp.float32)
    half = jnp.float32(0.5)
    threehalf = jnp.float32(1.5)
    for _ in range(3):
        y = y * (threehalf - half * z * y * y)
    return y


def _sc_all(idx2, tab128, xnT, params):
    mesh = plsc.VectorSubcoreMesh(core_axis_name="c", subcore_axis_name="s")

    @functools.partial(
        pl.kernel,
        out_type=jax.ShapeDtypeStruct((_O, _FC + _FN, _D, _B), jnp.float32),
        mesh=mesh,
        compiler_params=pltpu.CompilerParams(
            use_tc_tiling_on_sc=False, needs_layout_passes=False),
        scratch_types=[
            pltpu.VMEM((_FC, _BC), jnp.int32),
            pltpu.VMEM((_FC, _BC), jnp.int32),
            pltpu.VMEM((_CHUNKS := _BC, 128), jnp.float32),
            pltpu.VMEM((_D, _BC), jnp.float32),
            pltpu.VMEM((_FN, _BC), jnp.float32),
            pltpu.VMEM((8, 128), jnp.float32),
            pltpu.SemaphoreType.DMA,
        ],
    )
    def k(idx_hbm, tab_hbm, xn_hbm, par_hbm, out_hbm,
          idxc_v, idx4_v, packed_v, stage_v, xn_v, par_v, sem):
        wid = lax.axis_index("s") * _NC + lax.axis_index("c")
        iota = lax.iota(jnp.int32, _L)
        inv_d = jnp.float32(1.0 / _D)
        eps = jnp.float32(_EPS)

        pltpu.sync_copy(par_hbm, par_v)

        def splat(x):
            return jnp.full((_L,), x, jnp.int32)

        def par_splat(off):
            return plsc.load_gather(
                par_v, [splat(off >> 7), splat(off & 127)])

        def block(i, carry):
            blk = wid * _BLK_PER_W + i
            o = blk // jnp.int32(_NBC)
            bc = blk % jnp.int32(_NBC)

            pltpu.sync_copy(idx_hbm.at[pl.ds(blk * _FC, _FC)], idxc_v)
            pltpu.sync_copy(
                xn_hbm.at[:, pl.ds(o * _B + bc * _BC, _BC)], xn_v)

            def quarter(jj, c2):
                row = splat(jj // jnp.int32(8))
                col = splat((jj % jnp.int32(8)) * jnp.int32(_L)) + iota
                v = plsc.load_gather(idxc_v, [row, col])
                plsc.store_scatter(idx4_v, [row, col],
                                   lax.shift_right_logical(v, jnp.int32(2)))
                return c2

            lax.fori_loop(0, _FC * 8, quarter, 0)

            def cat_field(f, c2):
                pltpu.async_copy(tab_hbm.at[idx4_v.at[f]], packed_v,
                                 sem).wait()

                def sub(b16, c3):
                    lanes = splat(b16 * _L) + iota
                    colofs = (plsc.load_gather(idxc_v, [splat(f), lanes])
                              & jnp.int32(3)) * jnp.int32(_D)
                    vs = [plsc.load_gather(packed_v, [lanes, colofs + splat(d)])
                          for d in range(_D)]
                    s0 = (vs[0] + vs[4]) + (vs[8] + vs[12])
                    s1 = (vs[1] + vs[5]) + (vs[9] + vs[13])
                    s2 = (vs[2] + vs[6]) + (vs[10] + vs[14])
                    s3 = (vs[3] + vs[7]) + (vs[11] + vs[15])
                    s4 = (vs[16] + vs[20]) + (vs[24] + vs[28])
                    s5 = (vs[17] + vs[21]) + (vs[25] + vs[29])
                    s6 = (vs[18] + vs[22]) + (vs[26] + vs[30])
                    s7 = (vs[19] + vs[23]) + (vs[27] + vs[31])
                    sm = ((s0 + s1) + (s2 + s3)) + ((s4 + s5) + (s6 + s7))
                    q0 = sum([vs[j] * vs[j] for j in range(0, 8)],
                             jnp.zeros((_L,), jnp.float32))
                    q1 = sum([vs[j] * vs[j] for j in range(8, 16)],
                             jnp.zeros((_L,), jnp.float32))
                    q2 = sum([vs[j] * vs[j] for j in range(16, 24)],
                             jnp.zeros((_L,), jnp.float32))
                    q3 = sum([vs[j] * vs[j] for j in range(24, 32)],
                             jnp.zeros((_L,), jnp.float32))
                    sq = (q0 + q1) + (q2 + q3)
                    mean = sm * inv_d
                    var = sq * inv_d - mean * mean
                    rstd = _rsqrt_newton(var + eps)
                    for d in range(_D):
                        g = par_splat(_OFF_GC + d)
                        bb = par_splat(_OFF_BCZ + d)
                        y = (vs[d] - mean) * rstd * g + bb
                        stage_v[d, pl.ds(b16 * _L, _L)] = y
                    return c3

                lax.fori_loop(0, _NSUB, sub, 0)
                pltpu.sync_copy(
                    stage_v,
                    out_hbm.at[o, f, :, pl.ds(bc * _BC, _BC)])
                return c2

            lax.fori_loop(0, _FC, cat_field, 0)

            def num_field(f, c2):
                def sub(b16, c3):
                    x = xn_v[f, pl.ds(b16 * _L, _L)]
                    vs = []
                    for d in range(_D):
                        wv_off = f * jnp.int32(_D) + splat(_OFF_NW + d)
                        wv = plsc.load_gather(
                            par_v, [lax.shift_right_logical(
                                wv_off, jnp.int32(7)),
                                wv_off & jnp.int32(127)])
                        bv_off = f * jnp.int32(_D) + splat(_OFF_NB + d)
                        bv = plsc.load_gather(
                            par_v, [lax.shift_right_logical(
                                bv_off, jnp.int32(7)),
                                bv_off & jnp.int32(127)])
                        vs.append(x * wv + bv)
                    sm = sum(vs[1:], vs[0])
                    sq = sum([v * v for v in vs[1:]], vs[0] * vs[0])
                    mean = sm * inv_d
                    var = sq * inv_d - mean * mean
                    rstd = _rsqrt_newton(var + eps)
                    for d in range(_D):
                        g = par_splat(_OFF_GN + d)
                        bb = par_splat(_OFF_BN + d)
                        y = (vs[d] - mean) * rstd * g + bb
                        stage_v[d, pl.ds(b16 * _L, _L)] = y
                    return c3

                lax.fori_loop(0, _NSUB, sub, 0)
                pltpu.sync_copy(
                    stage_v,
                    out_hbm.at[o, jnp.int32(_FC) + f, :,
                               pl.ds(bc * _BC, _BC)])
                return c2

            lax.fori_loop(0, _FN, num_field, 0)
            return carry

        lax.fori_loop(0, _BLK_PER_W, block, 0)

    return k(idx2, tab128, xnT, params)


def kernel(x_cat, x_num, tables, num_w, num_b, cat_ln_g, cat_ln_b, num_ln_g,
           num_ln_b):
    B, O, Fc = x_cat.shape
    Fn = x_num.shape[-1]
    V, D = tables.shape[1], tables.shape[2]

    idx = x_cat.astype(jnp.int32) + jnp.arange(Fc, dtype=jnp.int32) * jnp.int32(V)
    idx = idx.reshape(B // _BC, _BC, O, Fc).transpose(2, 0, 3, 1)
    idx2 = idx.reshape(O * (B // _BC) * Fc, _BC)

    xnT = jnp.transpose(x_num, (2, 1, 0)).reshape(Fn, O * B)

    params = jnp.concatenate([
        num_w.reshape(-1), num_b.reshape(-1), cat_ln_g, cat_ln_b,
        num_ln_g, num_ln_b, jnp.zeros((64,), jnp.float32)]).reshape(8, 128)

    out = _sc_all(idx2, tables.reshape(Fc * V * D // 128, 128), xnT, params)
    return jnp.transpose(out, (3, 0, 1, 2))


# speedup vs baseline: 1.7885x; 1.1899x over previous
import functools

import jax
import jax.numpy as jnp
from jax import lax
from jax.experimental import pallas as pl
from jax.experimental.pallas import tpu as pltpu
from jax.experimental.pallas import tpu_sc as plsc

_NC = 2
_NS = 16
_NW = _NC * _NS
_L = 16
_EPS = 1e-5

_B = 1024
_O = 20
_FC = 26
_FN = 13
_D = 32
_BC = 128
_NBC = _B // _BC
_NBLK = _O * _NBC
_BLK_PER_W = _NBLK // _NW
_NSUB = _BC // _L

_OFF_NW = 0
_OFF_NB = _FN * _D
_OFF_GC = 2 * _FN * _D
_OFF_BCZ = _OFF_GC + _D
_OFF_GN = _OFF_GC + 2 * _D
_OFF_BN = _OFF_GC + 3 * _D


def _rsqrt_newton(z):
    i = jnp.int32(0x5F3759DF) - lax.shift_right_logical(
        lax.bitcast_convert_type(z, jnp.int32), jnp.int32(1))
    y = lax.bitcast_convert_type(i, jnp.float32)
    half = jnp.float32(0.5)
    threehalf = jnp.float32(1.5)
    for _ in range(3):
        y = y * (threehalf - half * z * y * y)
    return y


def _sc_all(idx2, tab, xnT, params):
    mesh = plsc.VectorSubcoreMesh(core_axis_name="c", subcore_axis_name="s")

    @functools.partial(
        pl.kernel,
        out_type=jax.ShapeDtypeStruct((_O, _FC + _FN, _D, _B), jnp.float32),
        mesh=mesh,
        compiler_params=pltpu.CompilerParams(
            use_tc_tiling_on_sc=False, needs_layout_passes=False),
        scratch_types=[
            pltpu.VMEM((_FC, _BC), jnp.int32),
            pltpu.VMEM((2, _BC, _D), jnp.float32),
            pltpu.VMEM((2, _D, _BC), jnp.float32),
            pltpu.VMEM((_FN, _BC), jnp.float32),
            pltpu.VMEM((8, 128), jnp.float32),
            pltpu.SemaphoreType.DMA,
            pltpu.SemaphoreType.DMA,
        ],
    )
    def k(idx_hbm, tab_hbm, xn_hbm, par_hbm, out_hbm,
          idxc_v, rows_v, stage_v, xn_v, par_v, sem_g, sem_o):
        wid = lax.axis_index("s") * _NC + lax.axis_index("c")
        iota = lax.iota(jnp.int32, _L)
        inv_d = jnp.float32(1.0 / _D)
        eps = jnp.float32(_EPS)

        pltpu.sync_copy(par_hbm, par_v)

        def splat(x):
            return jnp.full((_L,), x, jnp.int32)

        def par_splat(off):
            return plsc.load_gather(
                par_v, [splat(off >> 7), splat(off & 127)])

        def ln_store(vs, par, gofs, bofs):
            s0 = (vs[0] + vs[4]) + (vs[8] + vs[12])
            s1 = (vs[1] + vs[5]) + (vs[9] + vs[13])
            s2 = (vs[2] + vs[6]) + (vs[10] + vs[14])
            s3 = (vs[3] + vs[7]) + (vs[11] + vs[15])
            s4 = (vs[16] + vs[20]) + (vs[24] + vs[28])
            s5 = (vs[17] + vs[21]) + (vs[25] + vs[29])
            s6 = (vs[18] + vs[22]) + (vs[26] + vs[30])
            s7 = (vs[19] + vs[23]) + (vs[27] + vs[31])
            sm = ((s0 + s1) + (s2 + s3)) + ((s4 + s5) + (s6 + s7))
            zero = jnp.zeros((_L,), jnp.float32)
            q0 = sum([vs[j] * vs[j] for j in range(0, 8)], zero)
            q1 = sum([vs[j] * vs[j] for j in range(8, 16)], zero)
            q2 = sum([vs[j] * vs[j] for j in range(16, 24)], zero)
            q3 = sum([vs[j] * vs[j] for j in range(24, 32)], zero)
            sq = (q0 + q1) + (q2 + q3)
            mean = sm * inv_d
            var = sq * inv_d - mean * mean
            rstd = _rsqrt_newton(var + eps)
            ys = []
            for d in range(_D):
                g = par_splat(gofs + d)
                bb = par_splat(bofs + d)
                ys.append((vs[d] - mean) * rstd * g + bb)
            return ys

        def fire_gather(f, par):
            return pltpu.async_copy(
                tab_hbm.at[idxc_v.at[f]], rows_v.at[par], sem_g)

        def block(i, carry):
            blk = wid * _BLK_PER_W + i
            o = blk // jnp.int32(_NBC)
            bc = blk % jnp.int32(_NBC)

            pltpu.sync_copy(idx_hbm.at[pl.ds(blk * _FC, _FC)], idxc_v)
            pltpu.sync_copy(
                xn_hbm.at[:, pl.ds(o * _B + bc * _BC, _BC)], xn_v)

            fire_gather(jnp.int32(0), jnp.int32(0))

            def out_dma(fidx, par):
                return pltpu.async_copy(
                    stage_v.at[par],
                    out_hbm.at[o, fidx, :, pl.ds(bc * _BC, _BC)], sem_o)

            def wait_out_drain(fidx, par):
                pltpu.make_async_copy(
                    stage_v.at[par],
                    out_hbm.at[o, fidx, :, pl.ds(bc * _BC, _BC)],
                    sem_o).wait()

            def cat_field(f, c2):
                par = f % jnp.int32(2)
                pltpu.make_async_copy(
                    tab_hbm.at[idxc_v.at[f]], rows_v.at[par], sem_g).wait()

                @pl.when(f + 1 < _FC)
                def _():
                    fire_gather(f + 1, (f + 1) % jnp.int32(2))

                @pl.when(f >= 2)
                def _():
                    wait_out_drain(f, par)

                def sub(b16, c3):
                    lanes = splat(b16 * _L) + iota
                    vs = [plsc.load_gather(
                        rows_v, [splat(par), lanes, splat(d)])
                        for d in range(_D)]
                    ys = ln_store(vs, par, _OFF_GC, _OFF_BCZ)
                    for d in range(_D):
                        stage_v[par, d, pl.ds(b16 * _L, _L)] = ys[d]
                    return c3

                lax.fori_loop(0, _NSUB, sub, 0)
                out_dma(f, par)
                return c2

            lax.fori_loop(0, _FC, cat_field, 0)

            def num_field(f, c2):
                par = f % jnp.int32(2)

                @pl.when(f >= 2)
                def _():
                    wait_out_drain(f, par)

                def sub(b16, c3):
                    x = xn_v[f, pl.ds(b16 * _L, _L)]
                    vs = []
                    for d in range(_D):
                        wv_off = f * jnp.int32(_D) + splat(_OFF_NW + d)
                        wv = plsc.load_gather(
                            par_v, [lax.shift_right_logical(
                                wv_off, jnp.int32(7)),
                                wv_off & jnp.int32(127)])
                        bv_off = f * jnp.int32(_D) + splat(_OFF_NB + d)
                        bv = plsc.load_gather(
                            par_v, [lax.shift_right_logical(
                                bv_off, jnp.int32(7)),
                                bv_off & jnp.int32(127)])
                        vs.append(x * wv + bv)
                    ys = ln_store(vs, par, _OFF_GN, _OFF_BN)
                    for d in range(_D):
                        stage_v[par, d, pl.ds(b16 * _L, _L)] = ys[d]
                    return c3

                lax.fori_loop(0, _NSUB, sub, 0)
                out_dma(jnp.int32(_FC) + f, par)
                return c2

            wait_out_drain(jnp.int32(_FC - 2), jnp.int32(0))
            wait_out_drain(jnp.int32(_FC - 1), jnp.int32(1))
            lax.fori_loop(0, _FN, num_field, 0)
            wait_out_drain(jnp.int32(_FC + _FN - 2), jnp.int32(1))
            wait_out_drain(jnp.int32(_FC + _FN - 1), jnp.int32(0))
            return carry

        lax.fori_loop(0, _BLK_PER_W, block, 0)

    return k(idx2, tab, xnT, params)


def kernel(x_cat, x_num, tables, num_w, num_b, cat_ln_g, cat_ln_b, num_ln_g,
           num_ln_b):
    B, O, Fc = x_cat.shape
    Fn = x_num.shape[-1]
    V, D = tables.shape[1], tables.shape[2]

    idx = x_cat.astype(jnp.int32) + jnp.arange(Fc, dtype=jnp.int32) * jnp.int32(V)
    idx = idx.reshape(B // _BC, _BC, O, Fc).transpose(2, 0, 3, 1)
    idx2 = idx.reshape(O * (B // _BC) * Fc, _BC)

    xnT = jnp.transpose(x_num, (2, 1, 0)).reshape(Fn, O * B)

    params = jnp.concatenate([
        num_w.reshape(-1), num_b.reshape(-1), cat_ln_g, cat_ln_b,
        num_ln_g, num_ln_b, jnp.zeros((64,), jnp.float32)]).reshape(8, 128)

    out = _sc_all(idx2, tables.reshape(Fc * V, D), xnT, params)
    return jnp.transpose(out, (3, 0, 1, 2))


# speedup vs baseline: 1.8931x; 1.0585x over previous
import functools

import jax
import jax.numpy as jnp
from jax import lax
from jax.experimental import pallas as pl
from jax.experimental.pallas import tpu as pltpu
from jax.experimental.pallas import tpu_sc as plsc

_NC = 2
_NS = 16
_NW = _NC * _NS
_L = 16
_EPS = 1e-5

_B = 1024
_O = 20
_FC = 26
_FN = 13
_D = 32
_BC = 128
_NBC = _B // _BC
_NBLK = _O * _NBC
_BLK_PER_W = _NBLK // _NW
_NSUB = _BC // _L

_OFF_NW = 0
_OFF_NB = _FN * _D
_OFF_GC = 2 * _FN * _D
_OFF_BCZ = _OFF_GC + _D
_OFF_GN = _OFF_GC + 2 * _D
_OFF_BN = _OFF_GC + 3 * _D


def _rsqrt_newton(z):
    i = jnp.int32(0x5F3759DF) - lax.shift_right_logical(
        lax.bitcast_convert_type(z, jnp.int32), jnp.int32(1))
    y = lax.bitcast_convert_type(i, jnp.float32)
    half = jnp.float32(0.5)
    threehalf = jnp.float32(1.5)
    for _ in range(3):
        y = y * (threehalf - half * z * y * y)
    return y


def _sc_all(idx2, tab, xnT, params):
    mesh = plsc.VectorSubcoreMesh(core_axis_name="c", subcore_axis_name="s")

    @functools.partial(
        pl.kernel,
        out_type=jax.ShapeDtypeStruct((_O * (_FC + _FN) * _D, _B), jnp.float32),
        mesh=mesh,
        compiler_params=pltpu.CompilerParams(
            use_tc_tiling_on_sc=False, needs_layout_passes=False),
        scratch_types=[
            pltpu.VMEM((_FC, _BC), jnp.int32),
            pltpu.VMEM((2, _BC, _D), jnp.float32),
            pltpu.VMEM((2, _D, _BC), jnp.float32),
            pltpu.VMEM((_FN, _BC), jnp.float32),
            pltpu.VMEM((8, 128), jnp.float32),
            pltpu.SemaphoreType.DMA,
            pltpu.SemaphoreType.DMA,
        ],
    )
    def k(idx_hbm, tab_hbm, xn_hbm, par_hbm, out_hbm,
          idxc_v, rows_v, stage_v, xn_v, par_v, sem_g, sem_o):
        wid = lax.axis_index("s") * _NC + lax.axis_index("c")
        iota = lax.iota(jnp.int32, _L)
        inv_d = jnp.float32(1.0 / _D)
        eps = jnp.float32(_EPS)

        pltpu.sync_copy(par_hbm, par_v)

        def splat(x):
            return jnp.full((_L,), x, jnp.int32)

        def par_splat(off):
            return plsc.load_gather(
                par_v, [splat(off >> 7), splat(off & 127)])

        def ln_store(vs, par, gofs, bofs):
            s0 = (vs[0] + vs[4]) + (vs[8] + vs[12])
            s1 = (vs[1] + vs[5]) + (vs[9] + vs[13])
            s2 = (vs[2] + vs[6]) + (vs[10] + vs[14])
            s3 = (vs[3] + vs[7]) + (vs[11] + vs[15])
            s4 = (vs[16] + vs[20]) + (vs[24] + vs[28])
            s5 = (vs[17] + vs[21]) + (vs[25] + vs[29])
            s6 = (vs[18] + vs[22]) + (vs[26] + vs[30])
            s7 = (vs[19] + vs[23]) + (vs[27] + vs[31])
            sm = ((s0 + s1) + (s2 + s3)) + ((s4 + s5) + (s6 + s7))
            zero = jnp.zeros((_L,), jnp.float32)
            q0 = sum([vs[j] * vs[j] for j in range(0, 8)], zero)
            q1 = sum([vs[j] * vs[j] for j in range(8, 16)], zero)
            q2 = sum([vs[j] * vs[j] for j in range(16, 24)], zero)
            q3 = sum([vs[j] * vs[j] for j in range(24, 32)], zero)
            sq = (q0 + q1) + (q2 + q3)
            mean = sm * inv_d
            var = sq * inv_d - mean * mean
            rstd = _rsqrt_newton(var + eps)
            ys = []
            for d in range(_D):
                g = par_splat(gofs + d)
                bb = par_splat(bofs + d)
                ys.append((vs[d] - mean) * rstd * g + bb)
            return ys

        def fire_gather(f, par):
            return pltpu.async_copy(
                tab_hbm.at[f].at[idxc_v.at[f]], rows_v.at[par], sem_g)

        def block(i, carry):
            blk = wid * _BLK_PER_W + i
            o = blk // jnp.int32(_NBC)
            bc = blk % jnp.int32(_NBC)

            pltpu.sync_copy(idx_hbm.at[pl.ds(blk * _FC, _FC)], idxc_v)
            pltpu.sync_copy(
                xn_hbm.at[:, pl.ds(o * _B + bc * _BC, _BC)], xn_v)

            fire_gather(jnp.int32(0), jnp.int32(0))

            def out_dma(fidx, par):
                row0 = (o * jnp.int32(_FC + _FN) + fidx) * jnp.int32(_D)
                return pltpu.async_copy(
                    stage_v.at[par],
                    out_hbm.at[pl.ds(row0, _D), pl.ds(bc * _BC, _BC)], sem_o)

            def wait_out_drain(fidx, par):
                row0 = (o * jnp.int32(_FC + _FN) + fidx) * jnp.int32(_D)
                pltpu.make_async_copy(
                    stage_v.at[par],
                    out_hbm.at[pl.ds(row0, _D), pl.ds(bc * _BC, _BC)],
                    sem_o).wait()

            def cat_field(f, c2):
                par = f % jnp.int32(2)
                pltpu.make_async_copy(
                    tab_hbm.at[f].at[idxc_v.at[f]], rows_v.at[par],
                    sem_g).wait()

                @pl.when(f + 1 < _FC)
                def _():
                    fire_gather(f + 1, (f + 1) % jnp.int32(2))

                @pl.when(f >= 2)
                def _():
                    wait_out_drain(f, par)

                def sub(b8, c3):
                    for half in range(2):
                        b16 = b8 * 2 + half
                        lanes = splat(b16 * _L) + iota
                        vs = [plsc.load_gather(
                            rows_v, [splat(par), lanes, splat(d)])
                            for d in range(_D)]
                        ys = ln_store(vs, par, _OFF_GC, _OFF_BCZ)
                        for d in range(_D):
                            stage_v[par, d, pl.ds(b16 * _L, _L)] = ys[d]
                    return c3

                lax.fori_loop(0, _NSUB // 2, sub, 0)
                out_dma(f, par)
                return c2

            lax.fori_loop(0, _FC, cat_field, 0)

            def num_field(f, c2):
                par = f % jnp.int32(2)

                @pl.when(f >= 2)
                def _():
                    wait_out_drain(f, par)

                def sub(b8, c3):
                    wvs, bvs = [], []
                    for d in range(_D):
                        wv_off = f * jnp.int32(_D) + splat(_OFF_NW + d)
                        wvs.append(plsc.load_gather(
                            par_v, [lax.shift_right_logical(
                                wv_off, jnp.int32(7)),
                                wv_off & jnp.int32(127)]))
                        bv_off = f * jnp.int32(_D) + splat(_OFF_NB + d)
                        bvs.append(plsc.load_gather(
                            par_v, [lax.shift_right_logical(
                                bv_off, jnp.int32(7)),
                                bv_off & jnp.int32(127)]))
                    for half in range(2):
                        b16 = b8 * 2 + half
                        x = xn_v[f, pl.ds(b16 * _L, _L)]
                        vs = [x * wvs[d] + bvs[d] for d in range(_D)]
                        ys = ln_store(vs, par, _OFF_GN, _OFF_BN)
                        for d in range(_D):
                            stage_v[par, d, pl.ds(b16 * _L, _L)] = ys[d]
                    return c3

                lax.fori_loop(0, _NSUB // 2, sub, 0)
                out_dma(jnp.int32(_FC) + f, par)
                return c2

            wait_out_drain(jnp.int32(_FC - 2), jnp.int32(0))
            wait_out_drain(jnp.int32(_FC - 1), jnp.int32(1))
            lax.fori_loop(0, _FN, num_field, 0)
            wait_out_drain(jnp.int32(_FC + _FN - 2), jnp.int32(1))
            wait_out_drain(jnp.int32(_FC + _FN - 1), jnp.int32(0))
            return carry

        lax.fori_loop(0, _BLK_PER_W, block, 0)

    return k(idx2, tab, xnT, params)


def kernel(x_cat, x_num, tables, num_w, num_b, cat_ln_g, cat_ln_b, num_ln_g,
           num_ln_b):
    B, O, Fc = x_cat.shape
    Fn = x_num.shape[-1]
    V, D = tables.shape[1], tables.shape[2]

    idx = x_cat.astype(jnp.int32)
    idx = idx.reshape(B // _BC, _BC, O, Fc).transpose(2, 0, 3, 1)
    idx2 = idx.reshape(O * (B // _BC) * Fc, _BC)

    xnT = jnp.transpose(x_num, (2, 1, 0)).reshape(Fn, O * B)

    params = jnp.concatenate([
        num_w.reshape(-1), num_b.reshape(-1), cat_ln_g, cat_ln_b,
        num_ln_g, num_ln_b, jnp.zeros((64,), jnp.float32)]).reshape(8, 128)

    out = _sc_all(idx2, tables, xnT, params)
    return jnp.transpose(
        out.reshape(O, Fc + Fn, D, B), (3, 0, 1, 2))
